# Initial kernel scaffold; baseline (speedup 1.0000x reference)
import jax
import jax.numpy as jnp
from jax.experimental import pallas as pl


def kernel(points, W, gamma, beta):
    raise NotImplementedError("write your pallas kernel here")


# speedup vs baseline: 47.6913x; 47.6913x over previous
import functools

import jax
import jax.numpy as jnp
from jax import lax
from jax.experimental import pallas as pl
from jax.experimental.pallas import tpu as pltpu
from jax.experimental.pallas import tpu_sc as plsc

KNN_K = 16
EPS_BN = 1e-5
BIGF = 1e30

N_REAL = 10000
NPAD = 10240
QBLK = 128
PBLK = 1024
SBLK = 256
GCH = NPAD // 128
NTASK = KNN_K * GCH
NWORK = 32
TPW = NTASK // NWORK


def _proj_body(pts_ref, w1t_ref, wdt_ref, p1_ref, pd_ref):
    x = pts_ref[...]
    p1_ref[...] = jnp.dot(x, w1t_ref[...], preferred_element_type=jnp.float32)
    pd_ref[...] = jnp.dot(x, wdt_ref[...], preferred_element_type=jnp.float32)


def _knn_body(q_ref, xt_ref, idx_ref):
    q = q_ref[...]
    xt = xt_ref[...]
    sqq = jnp.sum(q * q, axis=1, keepdims=True)
    sqc = jnp.sum(xt * xt, axis=0, keepdims=True)
    dot = jnp.dot(q, xt, preferred_element_type=jnp.float32)
    d = (sqq - 2.0 * dot) + sqc
    col = lax.broadcasted_iota(jnp.int32, d.shape, 1)
    d = jnp.where(col < N_REAL, d, BIGF)
    for k in range(KNN_K):
        m = jnp.min(d, axis=1, keepdims=True)
        am = jnp.min(jnp.where(d == m, col, jnp.int32(2**30)), axis=1)
        idx_ref[k, :] = am
        d = jnp.where(col == am[:, None], BIGF, d)


def _pool_body(g_ref, pd_ref, mx_ref, mn_ref, s1_ref, s2_ref):
    i = pl.program_id(0)
    g = g_ref[...]
    pd = pd_ref[...]
    gmax = jnp.max(g, axis=0)
    gmin = jnp.min(g, axis=0)
    gsum = jnp.sum(g, axis=0)
    gsq = jnp.sum(g * g, axis=0)
    mx_ref[...] = gmax + pd
    mn_ref[...] = gmin + pd
    row = lax.broadcasted_iota(jnp.int32, (SBLK, 128), 0) + i * SBLK
    valid = row < N_REAL
    kf = float(KNN_K)
    p1 = jnp.sum(jnp.where(valid, gsum + kf * pd, 0.0), axis=0, keepdims=True)
    p2 = jnp.sum(
        jnp.where(valid, gsq + 2.0 * pd * gsum + kf * pd * pd, 0.0),
        axis=0, keepdims=True)

    @pl.when(i == 0)
    def _():
        s1_ref[...] = jnp.zeros_like(s1_ref)
        s2_ref[...] = jnp.zeros_like(s2_ref)

    s1_ref[...] += jnp.broadcast_to(p1, s1_ref.shape)
    s2_ref[...] += jnp.broadcast_to(p2, s2_ref.shape)


def _final_body(mx_ref, mn_ref, s1_ref, s2_ref, gam_ref, bet_ref, o_ref):
    cnt = float(N_REAL * KNN_K)
    mean = s1_ref[0:1, :] * (1.0 / cnt)
    ex2 = s2_ref[0:1, :] * (1.0 / cnt)
    var = ex2 - mean * mean
    scale = gam_ref[...] * lax.rsqrt(var + EPS_BN)
    shift = bet_ref[...] - mean * scale
    sel = jnp.where(scale >= 0.0, mx_ref[...], mn_ref[...])
    o_ref[...] = jnp.maximum(sel * scale + shift, 0.0)


def _sc_gather(p1, idx2):
    mesh = plsc.VectorSubcoreMesh(core_axis_name="c", subcore_axis_name="s")

    @functools.partial(
        pl.kernel,
        out_type=jax.ShapeDtypeStruct((NTASK, 128, 128), jnp.float32),
        mesh=mesh,
        scratch_types=[
            pltpu.VMEM((128,), jnp.int32),
            pltpu.VMEM((128, 128), jnp.float32),
            pltpu.SemaphoreType.DMA,
        ],
    )
    def gather_k(p1_hbm, idx_hbm, g_hbm, idx_v, rows_v, sem):
        wid = lax.axis_index("s") * 2 + lax.axis_index("c")

        def body(j, carry):
            t = wid * TPW + j
            pltpu.sync_copy(idx_hbm.at[t], idx_v)
            pltpu.async_copy(p1_hbm.at[idx_v], rows_v, sem).wait()
            pltpu.sync_copy(rows_v, g_hbm.at[t])
            return carry

        lax.fori_loop(0, TPW, body, 0)

    return gather_k(p1, idx2)


def kernel(points, W, gamma, beta):
    B, N, C = points.shape
    pts = points[0]
    ptsP = jnp.zeros((NPAD, C), jnp.float32).at[:N].set(pts)
    xyzP = jnp.zeros((NPAD, 8), jnp.float32).at[:N, :3].set(pts[:, -3:])
    xyzT = xyzP.T
    w1t = W[:, :C].T
    wdt = (W[:, C:] - W[:, :C]).T

    p1, pd = pl.pallas_call(
        _proj_body,
        grid=(NPAD // PBLK,),
        in_specs=[
            pl.BlockSpec((PBLK, C), lambda i: (i, 0)),
            pl.BlockSpec((C, C), lambda i: (0, 0)),
            pl.BlockSpec((C, C), lambda i: (0, 0)),
        ],
        out_specs=[
            pl.BlockSpec((PBLK, C), lambda i: (i, 0)),
            pl.BlockSpec((PBLK, C), lambda i: (i, 0)),
        ],
        out_shape=[
            jax.ShapeDtypeStruct((NPAD, C), jnp.float32),
            jax.ShapeDtypeStruct((NPAD, C), jnp.float32),
        ],
    )(ptsP, w1t, wdt)

    idxT = pl.pallas_call(
        _knn_body,
        grid=(NPAD // QBLK,),
        in_specs=[
            pl.BlockSpec((QBLK, 8), lambda i: (i, 0)),
            pl.BlockSpec((8, NPAD), lambda i: (0, 0)),
        ],
        out_specs=pl.BlockSpec((KNN_K, QBLK), lambda i: (0, i)),
        out_shape=jax.ShapeDtypeStruct((KNN_K, NPAD), jnp.int32),
    )(xyzP, xyzT)

    idx2 = idxT.reshape(NTASK, 128)
    g = _sc_gather(p1, idx2)
    g3 = g.reshape(KNN_K, NPAD, C)

    mx, mn, s1, s2 = pl.pallas_call(
        _pool_body,
        grid=(NPAD // SBLK,),
        in_specs=[
            pl.BlockSpec((KNN_K, SBLK, C), lambda i: (0, i, 0)),
            pl.BlockSpec((SBLK, C), lambda i: (i, 0)),
        ],
        out_specs=[
            pl.BlockSpec((SBLK, C), lambda i: (i, 0)),
            pl.BlockSpec((SBLK, C), lambda i: (i, 0)),
            pl.BlockSpec((8, C), lambda i: (0, 0)),
            pl.BlockSpec((8, C), lambda i: (0, 0)),
        ],
        out_shape=[
            jax.ShapeDtypeStruct((NPAD, C), jnp.float32),
            jax.ShapeDtypeStruct((NPAD, C), jnp.float32),
            jax.ShapeDtypeStruct((8, C), jnp.float32),
            jax.ShapeDtypeStruct((8, C), jnp.float32),
        ],
    )(g3, pd)

    out = pl.pallas_call(
        _final_body,
        grid=(NPAD // SBLK,),
        in_specs=[
            pl.BlockSpec((SBLK, C), lambda i: (i, 0)),
            pl.BlockSpec((SBLK, C), lambda i: (i, 0)),
            pl.BlockSpec((8, C), lambda i: (0, 0)),
            pl.BlockSpec((8, C), lambda i: (0, 0)),
            pl.BlockSpec((1, C), lambda i: (0, 0)),
            pl.BlockSpec((1, C), lambda i: (0, 0)),
        ],
        out_specs=pl.BlockSpec((SBLK, C), lambda i: (i, 0)),
        out_shape=jax.ShapeDtypeStruct((NPAD, C), jnp.float32),
    )(mx, mn, s1, s2, gamma[None, :], beta[None, :])

    return out[:N][None]


# speedup vs baseline: 55.7828x; 1.1697x over previous
import functools

import jax
import jax.numpy as jnp
from jax import lax
from jax.experimental import pallas as pl
from jax.experimental.pallas import tpu as pltpu
from jax.experimental.pallas import tpu_sc as plsc

KNN_K = 16
EPS_BN = 1e-5
BIGF = 1e30

N_REAL = 10000
NPAD = 10240
QBLK = 128
PBLK = 1024
SBLK = 256
GCH = NPAD // 128
NTASK = KNN_K * GCH
NWORK = 32
TPW = NTASK // NWORK


def _proj_body(pts_ref, w1t_ref, wdt_ref, p1_ref, pd_ref):
    x = pts_ref[...]
    p1_ref[...] = jnp.dot(x, w1t_ref[...], preferred_element_type=jnp.float32)
    pd_ref[...] = jnp.dot(x, wdt_ref[...], preferred_element_type=jnp.float32)


def _knn_body(q_ref, xt_ref, idx_ref):
    q = q_ref[...]
    xt = xt_ref[...]
    sqq = jnp.sum(q * q, axis=1, keepdims=True)
    sqc = jnp.sum(xt * xt, axis=0, keepdims=True)
    dot = jnp.dot(q, xt, preferred_element_type=jnp.float32)
    d = (sqq - 2.0 * dot) + sqc
    col = lax.broadcasted_iota(jnp.int32, d.shape, 1)
    d = jnp.where(col < N_REAL, d, BIGF)
    for k in range(KNN_K):
        m = jnp.min(d, axis=1, keepdims=True)
        eq = d == m
        am = jnp.min(jnp.where(eq, col, jnp.int32(2**30)), axis=1)
        idx_ref[k, :] = am
        d = jnp.where(eq, BIGF, d)


def _pool_body(g_ref, pd_ref, mx_ref, mn_ref, s1_ref, s2_ref):
    i = pl.program_id(0)
    g = g_ref[...]
    pd = pd_ref[...]
    gmax = jnp.max(g, axis=0)
    gmin = jnp.min(g, axis=0)
    gsum = jnp.sum(g, axis=0)
    gsq = jnp.sum(g * g, axis=0)
    mx_ref[...] = gmax + pd
    mn_ref[...] = gmin + pd
    row = lax.broadcasted_iota(jnp.int32, (SBLK, 128), 0) + i * SBLK
    valid = row < N_REAL
    kf = float(KNN_K)
    p1 = jnp.sum(jnp.where(valid, gsum + kf * pd, 0.0), axis=0, keepdims=True)
    p2 = jnp.sum(
        jnp.where(valid, gsq + 2.0 * pd * gsum + kf * pd * pd, 0.0),
        axis=0, keepdims=True)

    @pl.when(i == 0)
    def _():
        s1_ref[...] = jnp.zeros_like(s1_ref)
        s2_ref[...] = jnp.zeros_like(s2_ref)

    s1_ref[...] += jnp.broadcast_to(p1, s1_ref.shape)
    s2_ref[...] += jnp.broadcast_to(p2, s2_ref.shape)


def _final_body(mx_ref, mn_ref, s1_ref, s2_ref, gam_ref, bet_ref, o_ref):
    cnt = float(N_REAL * KNN_K)
    mean = s1_ref[0:1, :] * (1.0 / cnt)
    ex2 = s2_ref[0:1, :] * (1.0 / cnt)
    var = ex2 - mean * mean
    scale = gam_ref[...] * lax.rsqrt(var + EPS_BN)
    shift = bet_ref[...] - mean * scale
    sel = jnp.where(scale >= 0.0, mx_ref[...], mn_ref[...])
    o_ref[...] = jnp.maximum(sel * scale + shift, 0.0)


def _sc_gather(p1, idx2):
    mesh = plsc.VectorSubcoreMesh(core_axis_name="c", subcore_axis_name="s")

    @functools.partial(
        pl.kernel,
        out_type=jax.ShapeDtypeStruct((NTASK, 128, 128), jnp.float32),
        mesh=mesh,
        scratch_types=[
            pltpu.VMEM((2, 128), jnp.int32),
            pltpu.VMEM((2, 128, 128), jnp.float32),
            pltpu.SemaphoreType.DMA,
            pltpu.SemaphoreType.DMA,
        ],
    )
    def gather_k(p1_hbm, idx_hbm, g_hbm, idx_v, rows_v, semg, semw):
        wid = lax.axis_index("s") * 2 + lax.axis_index("c")
        base = wid * TPW

        pltpu.sync_copy(idx_hbm.at[base], idx_v.at[0])
        g_cp = pltpu.async_copy(p1_hbm.at[idx_v.at[0]], rows_v.at[0], semg)
        w_cp = None
        for j in range(TPW):
            b = j % 2
            g_cp.wait()
            if w_cp is not None:
                w_cp.wait()
            w_cp = pltpu.async_copy(rows_v.at[b], g_hbm.at[base + j], semw)
            if j + 1 < TPW:
                nb = (j + 1) % 2
                pltpu.sync_copy(idx_hbm.at[base + j + 1], idx_v.at[nb])
                g_cp = pltpu.async_copy(
                    p1_hbm.at[idx_v.at[nb]], rows_v.at[nb], semg)
        w_cp.wait()

    return gather_k(p1, idx2)


def kernel(points, W, gamma, beta):
    B, N, C = points.shape
    pts = points[0]
    ptsP = jnp.zeros((NPAD, C), jnp.float32).at[:N].set(pts)
    xyzP = jnp.zeros((NPAD, 8), jnp.float32).at[:N, :3].set(pts[:, -3:])
    xyzT = xyzP.T
    w1t = W[:, :C].T
    wdt = (W[:, C:] - W[:, :C]).T

    p1, pd = pl.pallas_call(
        _proj_body,
        grid=(NPAD // PBLK,),
        in_specs=[
            pl.BlockSpec((PBLK, C), lambda i: (i, 0)),
            pl.BlockSpec((C, C), lambda i: (0, 0)),
            pl.BlockSpec((C, C), lambda i: (0, 0)),
        ],
        out_specs=[
            pl.BlockSpec((PBLK, C), lambda i: (i, 0)),
            pl.BlockSpec((PBLK, C), lambda i: (i, 0)),
        ],
        out_shape=[
            jax.ShapeDtypeStruct((NPAD, C), jnp.float32),
            jax.ShapeDtypeStruct((NPAD, C), jnp.float32),
        ],
    )(ptsP, w1t, wdt)

    idxT = pl.pallas_call(
        _knn_body,
        grid=(NPAD // QBLK,),
        in_specs=[
            pl.BlockSpec((QBLK, 8), lambda i: (i, 0)),
            pl.BlockSpec((8, NPAD), lambda i: (0, 0)),
        ],
        out_specs=pl.BlockSpec((KNN_K, QBLK), lambda i: (0, i)),
        out_shape=jax.ShapeDtypeStruct((KNN_K, NPAD), jnp.int32),
    )(xyzP, xyzT)

    idx2 = idxT.reshape(NTASK, 128)
    g = _sc_gather(p1, idx2)
    g3 = g.reshape(KNN_K, NPAD, C)

    mx, mn, s1, s2 = pl.pallas_call(
        _pool_body,
        grid=(NPAD // SBLK,),
        in_specs=[
            pl.BlockSpec((KNN_K, SBLK, C), lambda i: (0, i, 0)),
            pl.BlockSpec((SBLK, C), lambda i: (i, 0)),
        ],
        out_specs=[
            pl.BlockSpec((SBLK, C), lambda i: (i, 0)),
            pl.BlockSpec((SBLK, C), lambda i: (i, 0)),
            pl.BlockSpec((8, C), lambda i: (0, 0)),
            pl.BlockSpec((8, C), lambda i: (0, 0)),
        ],
        out_shape=[
            jax.ShapeDtypeStruct((NPAD, C), jnp.float32),
            jax.ShapeDtypeStruct((NPAD, C), jnp.float32),
            jax.ShapeDtypeStruct((8, C), jnp.float32),
            jax.ShapeDtypeStruct((8, C), jnp.float32),
        ],
    )(g3, pd)

    out = pl.pallas_call(
        _final_body,
        grid=(NPAD // SBLK,),
        in_specs=[
            pl.BlockSpec((SBLK, C), lambda i: (i, 0)),
            pl.BlockSpec((SBLK, C), lambda i: (i, 0)),
            pl.BlockSpec((8, C), lambda i: (0, 0)),
            pl.BlockSpec((8, C), lambda i: (0, 0)),
            pl.BlockSpec((1, C), lambda i: (0, 0)),
            pl.BlockSpec((1, C), lambda i: (0, 0)),
        ],
        out_specs=pl.BlockSpec((SBLK, C), lambda i: (i, 0)),
        out_shape=jax.ShapeDtypeStruct((NPAD, C), jnp.float32),
    )(mx, mn, s1, s2, gamma[None, :], beta[None, :])

    return out[:N][None]
